# baseline (device time: 32510 ns/iter reference)
import jax
import jax.numpy as jnp
from jax import lax
from jax.experimental import pallas as pl
from jax.experimental.pallas import tpu as pltpu

N_DEV = 4
N_EXP = 8
CAP = 204.0


def kernel(x, router_W, route_idx, expert_W):
    m, d = x.shape
    e_loc, _, h = expert_W.shape
    r2d = route_idx.reshape(4, 128)

    def body(x_ref, ridx_ref, r2d_ref, ew_ref, out_ref,
             w_comm, r_comm, w_all, w_send, w_recv, r_send, r_recv):
        my = lax.axis_index("i")
        left = lax.rem(my + N_DEV - 1, N_DEV)
        right = lax.rem(my + 1, N_DEV)

        barrier = pltpu.get_barrier_semaphore()
        for nbr in (left, right):
            pl.semaphore_signal(barrier, inc=1, device_id=(nbr,),
                                device_id_type=pl.DeviceIdType.MESH)
        pl.semaphore_wait(barrier, 2)

        w_comm[0, :, :, :] = ew_ref[:, :, :].astype(jnp.bfloat16)
        r_comm[0, :, :] = r2d_ref[:, :]
        w_all[pl.ds(2 * my, 2), :, :] = w_comm[0, :, :, :]

        for hp in range(N_DEV - 1):
            w_rdma = pltpu.make_async_remote_copy(
                src_ref=w_comm.at[hp], dst_ref=w_comm.at[hp + 1],
                send_sem=w_send.at[hp], recv_sem=w_recv.at[hp],
                device_id=(right,), device_id_type=pl.DeviceIdType.MESH)
            r_rdma = pltpu.make_async_remote_copy(
                src_ref=r_comm.at[hp], dst_ref=r_comm.at[hp + 1],
                send_sem=r_send.at[hp], recv_sem=r_recv.at[hp],
                device_id=(right,), device_id_type=pl.DeviceIdType.MESH)
            w_rdma.start()
            r_rdma.start()
            w_rdma.wait()
            r_rdma.wait()
            origin = lax.rem(my - hp - 1 + N_DEV, N_DEV)
            w_all[pl.ds(2 * origin, 2), :, :] = w_comm[hp + 1, :, :, :]

        x_bf = x_ref[:, :].astype(jnp.bfloat16)
        route = ridx_ref[:, :]
        row = lax.broadcasted_iota(jnp.int32, (m, m), 0)
        col = lax.broadcasted_iota(jnp.int32, (m, m), 1)
        lower = (row >= col).astype(jnp.float32)
        eids = lax.broadcasted_iota(jnp.int32, (m, N_EXP), 1)
        onehot = (route == eids).astype(jnp.float32)
        rank = jnp.dot(lower, onehot, preferred_element_type=jnp.float32)

        acc = jnp.zeros((m, h), jnp.float32)
        for e in range(N_EXP):
            prefix = jnp.float32(0.0)
            for p in range(1, N_DEV):
                cnt = jnp.sum((r_comm[p, :, :] == e).astype(jnp.float32))
                prefix = prefix + cnt * (my >= p).astype(jnp.float32)
            ok = (route == e) & (rank[:, e:e + 1] + prefix <= CAP)
            xm = x_bf * ok.astype(jnp.bfloat16)
            acc = acc + jnp.dot(xm, w_all[e, :, :],
                                preferred_element_type=jnp.float32)
        out_ref[:, :] = acc

    return pl.pallas_call(
        body,
        out_shape=jax.ShapeDtypeStruct((m, h), jnp.float32),
        in_specs=[pl.BlockSpec(memory_space=pltpu.VMEM)] * 4,
        out_specs=pl.BlockSpec(memory_space=pltpu.VMEM),
        scratch_shapes=[
            pltpu.VMEM((N_DEV, e_loc, d, h), jnp.bfloat16),
            pltpu.VMEM((N_DEV, 4, 128), jnp.int32),
            pltpu.VMEM((N_EXP, d, h), jnp.bfloat16),
            pltpu.SemaphoreType.DMA((N_DEV - 1,)),
            pltpu.SemaphoreType.DMA((N_DEV - 1,)),
            pltpu.SemaphoreType.DMA((N_DEV - 1,)),
            pltpu.SemaphoreType.DMA((N_DEV - 1,)),
        ],
        compiler_params=pltpu.CompilerParams(collective_id=0),
    )(x, route_idx, r2d, expert_W)


# device time: 24554 ns/iter; 1.3240x vs baseline; 1.3240x over previous
import jax
import jax.numpy as jnp
from jax import lax
from jax.experimental import pallas as pl
from jax.experimental.pallas import tpu as pltpu

N_DEV = 4
N_EXP = 8
CAP = 204.0


def kernel(x, router_W, route_idx, expert_W):
    m, d = x.shape
    e_loc, _, h = expert_W.shape
    r2d = route_idx.reshape(4, 128)

    def body(x_ref, ridx_ref, r2d_ref, ew_ref, out_ref,
             w_peer, r_peer, w_send, w_recv, r_send, r_recv):
        my = lax.axis_index("i")

        barrier = pltpu.get_barrier_semaphore()
        for rel in (1, 2, 3):
            t = lax.rem(my + rel, N_DEV)
            pl.semaphore_signal(barrier, inc=1, device_id=(t,),
                                device_id_type=pl.DeviceIdType.MESH)
        pl.semaphore_wait(barrier, 3)

        w_peer[0, :, :, :] = ew_ref[:, :, :].astype(jnp.bfloat16)
        r_peer[0, :, :] = r2d_ref[:, :]

        sends = []
        for rel in (1, 2, 3):
            t = lax.rem(my + rel, N_DEV)
            j = N_DEV - rel
            for buf, ssem, rsem in ((w_peer, w_send, w_recv),
                                    (r_peer, r_send, r_recv)):
                rdma = pltpu.make_async_remote_copy(
                    src_ref=buf.at[0], dst_ref=buf.at[j],
                    send_sem=ssem.at[rel], recv_sem=rsem.at[j],
                    device_id=(t,), device_id_type=pl.DeviceIdType.MESH)
                rdma.start()
                sends.append(rdma)

        def recv_desc(buf, rsem, j):
            return pltpu.make_async_remote_copy(
                src_ref=buf.at[0], dst_ref=buf.at[j],
                send_sem=w_send.at[0], recv_sem=rsem.at[j],
                device_id=(my,), device_id_type=pl.DeviceIdType.MESH)

        x_bf = x_ref[:, :].astype(jnp.bfloat16)
        route = ridx_ref[:, :]
        row = lax.broadcasted_iota(jnp.int32, (m, m), 0)
        col = lax.broadcasted_iota(jnp.int32, (m, m), 1)
        lower = (row >= col).astype(jnp.float32)
        eids = lax.broadcasted_iota(jnp.int32, (m, N_EXP), 1)
        onehot = (route == eids).astype(jnp.float32)
        rank = jnp.dot(lower, onehot, preferred_element_type=jnp.float32)

        for j in (1, 3, 2):
            recv_desc(r_peer, r_recv, j).wait_recv()

        acc = jnp.zeros((m, h), jnp.float32)

        def slot_contrib(j, acc):
            o = lax.rem(my + j, N_DEV)
            for jj in range(e_loc):
                e = e_loc * o + jj
                is_e = route == e
                rank_e = jnp.sum(rank * (eids == e).astype(jnp.float32),
                                 axis=1, keepdims=True)
                prefix = jnp.float32(0.0)
                for p in range(1, N_DEV):
                    cnt = jnp.sum((r_peer[p, :, :] == e)
                                  .astype(jnp.float32))
                    prefix = prefix + cnt * (my + p >= N_DEV
                                             ).astype(jnp.float32)
                ok = is_e & (rank_e + prefix <= CAP)
                xm = x_bf * ok.astype(jnp.bfloat16)
                acc = acc + jnp.dot(xm, w_peer[j, jj, :, :],
                                    preferred_element_type=jnp.float32)
            return acc

        acc = slot_contrib(0, acc)
        for j in (1, 3, 2):
            recv_desc(w_peer, w_recv, j).wait_recv()
            acc = slot_contrib(j, acc)

        for rdma in sends:
            rdma.wait_send()
        out_ref[:, :] = acc

    return pl.pallas_call(
        body,
        out_shape=jax.ShapeDtypeStruct((m, h), jnp.float32),
        in_specs=[pl.BlockSpec(memory_space=pltpu.VMEM)] * 4,
        out_specs=pl.BlockSpec(memory_space=pltpu.VMEM),
        scratch_shapes=[
            pltpu.VMEM((N_DEV, e_loc, d, h), jnp.bfloat16),
            pltpu.VMEM((N_DEV, 4, 128), jnp.int32),
            pltpu.SemaphoreType.DMA((N_DEV,)),
            pltpu.SemaphoreType.DMA((N_DEV,)),
            pltpu.SemaphoreType.DMA((N_DEV,)),
            pltpu.SemaphoreType.DMA((N_DEV,)),
        ],
        compiler_params=pltpu.CompilerParams(collective_id=0),
    )(x, route_idx, r2d, expert_W)
